# baseline (speedup 1.0000x reference)
import functools

import jax
import jax.numpy as jnp
from jax import lax
from jax.experimental import pallas as pl
from jax.experimental.pallas import tpu as pltpu
from jax.experimental.pallas import tpu_sc as plsc

B = 128
V = 100000
HIST = 200
PRANGE = 100
L = 16
NIDX = 112
NC, NS = 2, 16
NW = NC * NS
ROWS_PER_W = B // NW
CHUNK = 25000
NCH = V // CHUNK


def _body(logits_hbm, idx_hbm, pen_hbm, out_hbm, buf, idxv, penv):
    wid = lax.axis_index("s") * NC + lax.axis_index("c")
    pltpu.sync_copy(pen_hbm, penv)
    pen = penv[...]
    for rr in range(ROWS_PER_W):
        r = wid * ROWS_PER_W + rr
        pltpu.sync_copy(idx_hbm.at[pl.ds(r * NIDX, NIDX)], idxv)
        for c in range(NCH):
            lo = c * CHUNK
            pltpu.sync_copy(logits_hbm.at[pl.ds(r * V + lo, CHUNK)], buf)
            gathered = []
            for k in range(NIDX // L):
                iv = idxv[pl.ds(k * L, L)]
                m = (iv >= lo) & (iv < lo + CHUNK)
                loc = jnp.where(m, iv - lo, 0)
                g = plsc.load_gather(buf, [loc], mask=m)
                gathered.append((loc, m, g))
            for loc, m, g in gathered:
                plsc.store_scatter(buf, [loc], g * pen, mask=m)
            pltpu.sync_copy(buf, out_hbm.at[pl.ds(r * V + lo, CHUNK)])


def kernel(logits, save_id, penalty_value, penality_range):
    del penality_range
    tgt = save_id[:, HIST - PRANGE:]
    idx_pad = jnp.concatenate([tgt, tgt[:, : NIDX - PRANGE]], axis=1)
    idx_pad = idx_pad.astype(jnp.int32).reshape(B * NIDX)
    pen16 = jnp.broadcast_to(penalty_value.astype(jnp.float32), (L,))

    run = functools.partial(
        pl.kernel,
        out_type=jax.ShapeDtypeStruct((B * V,), jnp.float32),
        mesh=plsc.VectorSubcoreMesh(
            core_axis_name="c", subcore_axis_name="s",
            num_cores=NC, num_subcores=NS,
        ),
        scratch_types=[
            pltpu.VMEM((CHUNK,), jnp.float32),
            pltpu.VMEM((NIDX,), jnp.int32),
            pltpu.VMEM((L,), jnp.float32),
        ],
        compiler_params=pltpu.CompilerParams(needs_layout_passes=False),
    )(_body)
    return run(logits.reshape(B * V), idx_pad, pen16).reshape(B, V)


# speedup vs baseline: 1.0480x; 1.0480x over previous
import functools

import jax
import jax.numpy as jnp
from jax import lax
from jax.experimental import pallas as pl
from jax.experimental.pallas import tpu as pltpu
from jax.experimental.pallas import tpu_sc as plsc

B = 128
V = 100000
HIST = 200
PRANGE = 100
L = 16
NIDX = 112
NC, NS = 2, 16
NW = NC * NS
ROWS_PER_W = B // NW
CHUNK = 25000
NCH = V // CHUNK


NBUF = 4
LA = 2


def _body(logits_hbm, idx_hbm, pen_hbm, out_hbm, *refs):
    bufs = refs[:NBUF]
    idxv, penv = refs[NBUF], refs[NBUF + 1]
    insems = refs[NBUF + 2: 2 * NBUF + 2]
    outsems = refs[2 * NBUF + 2: 3 * NBUF + 2]

    wid = lax.axis_index("s") * NC + lax.axis_index("c")
    pltpu.sync_copy(pen_hbm, penv)
    pltpu.sync_copy(
        idx_hbm.at[pl.ds(wid * ROWS_PER_W * NIDX, ROWS_PER_W * NIDX)], idxv)
    pen = penv[...]

    T = ROWS_PER_W * NCH

    def src(t):
        rr, c = divmod(t, NCH)
        return pl.ds((wid * ROWS_PER_W + rr) * V + c * CHUNK, CHUNK)

    ins = [None] * T
    outs = [None] * T
    for j in range(min(LA, T)):
        ins[j] = pltpu.async_copy(logits_hbm.at[src(j)], bufs[j % NBUF],
                                  insems[j % NBUF])
    for t in range(T):
        b = t % NBUF
        ins[t].wait()
        rr, c = divmod(t, NCH)
        lo = c * CHUNK
        gathered = []
        for k in range(NIDX // L):
            iv = idxv[pl.ds(rr * NIDX + k * L, L)]
            m = (iv >= lo) & (iv < lo + CHUNK)
            loc = jnp.where(m, iv - lo, 0)
            g = plsc.load_gather(bufs[b], [loc], mask=m)
            gathered.append((loc, m, g))
        for loc, m, g in gathered:
            plsc.store_scatter(bufs[b], [loc], g * pen, mask=m)
        outs[t] = pltpu.async_copy(bufs[b], out_hbm.at[src(t)], outsems[b])
        nxt = t + LA
        if nxt < T:
            nb = nxt % NBUF
            if nxt - NBUF >= 0:
                outs[nxt - NBUF].wait()
            ins[nxt] = pltpu.async_copy(logits_hbm.at[src(nxt)], bufs[nb],
                                        insems[nb])
    for t in range(max(0, T - NBUF), T):
        outs[t].wait()


def kernel(logits, save_id, penalty_value, penality_range):
    del penality_range
    tgt = save_id[:, HIST - PRANGE:]
    idx_pad = jnp.concatenate([tgt, tgt[:, : NIDX - PRANGE]], axis=1)
    idx_pad = idx_pad.astype(jnp.int32).reshape(B * NIDX)
    pen16 = jnp.broadcast_to(penalty_value.astype(jnp.float32), (L,))

    run = functools.partial(
        pl.kernel,
        out_type=jax.ShapeDtypeStruct((B * V,), jnp.float32),
        mesh=plsc.VectorSubcoreMesh(
            core_axis_name="c", subcore_axis_name="s",
            num_cores=NC, num_subcores=NS,
        ),
        scratch_types=(
            [pltpu.VMEM((CHUNK,), jnp.float32)] * NBUF
            + [pltpu.VMEM((ROWS_PER_W * NIDX,), jnp.int32),
               pltpu.VMEM((L,), jnp.float32)]
            + [pltpu.SemaphoreType.DMA] * (2 * NBUF)
        ),
        compiler_params=pltpu.CompilerParams(needs_layout_passes=False),
    )(_body)
    return run(logits.reshape(B * V), idx_pad, pen16).reshape(B, V)


# speedup vs baseline: 1.9498x; 1.8606x over previous
import functools

import jax
import jax.numpy as jnp
from jax import lax
from jax.experimental import pallas as pl
from jax.experimental.pallas import tpu as pltpu
from jax.experimental.pallas import tpu_sc as plsc

B = 128
V = 100000
HIST = 200
PRANGE = 100
L = 16
NIDX = 112
NC, NS = 2, 16
NW = NC * NS
RG = 8
NG = B // RG
WC = 3328
NFULL = 30
TAILC = V - NFULL * WC
NCHW = NFULL // 2
NBUF = 4
LA = 2


def _penalize(buf, idxv, pen, lo, width):
    def row_body(j, carry):
        gathered = []
        rowv = jnp.zeros((L,), jnp.int32) + j
        for k in range(NIDX // L):
            iv = idxv[pl.ds(j * NIDX + k * L, L)]
            m = (iv >= lo) & (iv < lo + width)
            colv = jnp.where(m, iv - lo, 0)
            g = plsc.load_gather(buf, [rowv, colv], mask=m)
            gathered.append((colv, m, g))
        for colv, m, g in gathered:
            plsc.store_scatter(buf, [rowv, colv], g * pen, mask=m)
        return carry
    lax.fori_loop(0, RG, row_body, 0)


def _body(logits_hbm, idx_hbm, pen_hbm, out_hbm, *refs):
    bufs = refs[:NBUF]
    tailbuf = refs[NBUF]
    idxv, penv = refs[NBUF + 1], refs[NBUF + 2]
    insems = refs[NBUF + 3: 2 * NBUF + 3]
    outsems = refs[2 * NBUF + 3: 3 * NBUF + 3]

    wid = lax.axis_index("s") * NC + lax.axis_index("c")
    g = wid // 2
    h = wid % 2
    r0 = pl.multiple_of(g * RG, RG)

    pltpu.sync_copy(pen_hbm, penv)
    pltpu.sync_copy(idx_hbm.at[pl.ds(g * RG * NIDX, RG * NIDX)], idxv)
    pen = penv[...]

    def col_lo(t):
        return (2 * t + h) * WC

    def hbm_slice(ref, t):
        return ref.at[pl.ds(r0, RG), pl.ds(pl.multiple_of(col_lo(t), 128), WC)]

    ins = [None] * NCHW
    outs = [None] * NCHW
    for j in range(min(LA, NCHW)):
        ins[j] = pltpu.async_copy(hbm_slice(logits_hbm, j), bufs[j % NBUF],
                                  insems[j % NBUF])
    for t in range(NCHW):
        b = t % NBUF
        ins[t].wait()
        _penalize(bufs[b], idxv, pen, col_lo(t), WC)
        outs[t] = pltpu.async_copy(bufs[b], hbm_slice(out_hbm, t), outsems[b])
        nxt = t + LA
        if nxt < NCHW:
            nb = nxt % NBUF
            if nxt - NBUF >= 0:
                outs[nxt - NBUF].wait()
            ins[nxt] = pltpu.async_copy(hbm_slice(logits_hbm, nxt), bufs[nb],
                                        insems[nb])
    @pl.when(h == 0)
    def _tail():
        tlo = NFULL * WC
        pltpu.sync_copy(
            logits_hbm.at[pl.ds(r0, RG), pl.ds(tlo, TAILC)], tailbuf)
        _penalize(tailbuf, idxv, pen, tlo, TAILC)
        pltpu.sync_copy(
            tailbuf, out_hbm.at[pl.ds(r0, RG), pl.ds(tlo, TAILC)])
    for t in range(max(0, NCHW - NBUF), NCHW):
        outs[t].wait()


def kernel(logits, save_id, penalty_value, penality_range):
    del penality_range
    tgt = save_id[:, HIST - PRANGE:]
    idx_pad = jnp.concatenate([tgt, tgt[:, : NIDX - PRANGE]], axis=1)
    idx_pad = idx_pad.astype(jnp.int32).reshape(B * NIDX)
    pen16 = jnp.broadcast_to(penalty_value.astype(jnp.float32), (L,))

    run = functools.partial(
        pl.kernel,
        out_type=jax.ShapeDtypeStruct((B, V), jnp.float32),
        mesh=plsc.VectorSubcoreMesh(
            core_axis_name="c", subcore_axis_name="s",
            num_cores=NC, num_subcores=NS,
        ),
        scratch_types=(
            [pltpu.VMEM((RG, WC), jnp.float32)] * NBUF
            + [pltpu.VMEM((RG, TAILC), jnp.float32),
               pltpu.VMEM((RG * NIDX,), jnp.int32),
               pltpu.VMEM((L,), jnp.float32)]
            + [pltpu.SemaphoreType.DMA] * (2 * NBUF)
        ),
        compiler_params=pltpu.CompilerParams(needs_layout_passes=False),
    )(_body)
    return run(logits, idx_pad, pen16)


# speedup vs baseline: 1.9522x; 1.0012x over previous
import functools

import jax
import jax.numpy as jnp
from jax import lax
from jax.experimental import pallas as pl
from jax.experimental.pallas import tpu as pltpu
from jax.experimental.pallas import tpu_sc as plsc

B = 128
V = 100000
HIST = 200
PRANGE = 100
L = 16
NIDX = 112
NC, NS = 2, 16
NW = NC * NS
RG = 8
NG = B // RG
WC = 3328
NFULL = 30
TAILC = V - NFULL * WC
NCHW = NFULL // 2
NBUF = 4
LA = 2


def _penalize(buf, idxv, pen, lo, width):
    def row_body(j, carry):
        gathered = []
        rowv = jnp.zeros((L,), jnp.int32) + j
        for k in range(NIDX // L):
            iv = idxv[pl.ds(j * NIDX + k * L, L)]
            m = (iv >= lo) & (iv < lo + width)
            colv = jnp.where(m, iv - lo, 0)
            g = plsc.load_gather(buf, [rowv, colv], mask=m)
            gathered.append((colv, m, g))
        for colv, m, g in gathered:
            plsc.store_scatter(buf, [rowv, colv], g * pen, mask=m)
        return carry
    lax.fori_loop(0, RG, row_body, 0)


def _body(logits_hbm, idx_hbm, pen_hbm, out_hbm, *refs):
    bufs = refs[:NBUF]
    tailbuf = refs[NBUF]
    idxv, penv = refs[NBUF + 1], refs[NBUF + 2]
    insems = refs[NBUF + 3: 2 * NBUF + 3]
    outsems = refs[2 * NBUF + 3: 3 * NBUF + 3]

    wid = lax.axis_index("s") * NC + lax.axis_index("c")
    g = wid // 2
    h = wid % 2
    r0 = pl.multiple_of(g * RG, RG)

    pltpu.sync_copy(pen_hbm, penv)
    pltpu.sync_copy(idx_hbm.at[pl.ds(g * RG * NIDX, RG * NIDX)], idxv)
    pen = penv[...]

    def col_lo(t):
        return (2 * t + h) * WC

    def hbm_slice(ref, t):
        return ref.at[pl.ds(r0, RG), pl.ds(pl.multiple_of(col_lo(t), 128), WC)]

    ins = [None] * NCHW
    outs = [None] * NCHW
    for j in range(min(LA, NCHW)):
        ins[j] = pltpu.async_copy(hbm_slice(logits_hbm, j), bufs[j % NBUF],
                                  insems[j % NBUF])
    for t in range(NCHW):
        b = t % NBUF
        ins[t].wait()
        _penalize(bufs[b], idxv, pen, col_lo(t), WC)
        outs[t] = pltpu.async_copy(bufs[b], hbm_slice(out_hbm, t), outsems[b])
        nxt = t + LA
        if nxt < NCHW:
            nb = nxt % NBUF
            if nxt - NBUF >= 0:
                outs[nxt - NBUF].wait()
            ins[nxt] = pltpu.async_copy(hbm_slice(logits_hbm, nxt), bufs[nb],
                                        insems[nb])
    @pl.when(h == 0)
    def _tail():
        tlo = NFULL * WC
        pltpu.sync_copy(
            logits_hbm.at[pl.ds(r0, RG), pl.ds(tlo, TAILC)], tailbuf)
        _penalize(tailbuf, idxv, pen, tlo, TAILC)
        pltpu.sync_copy(
            tailbuf, out_hbm.at[pl.ds(r0, RG), pl.ds(tlo, TAILC)])
    for t in range(max(0, NCHW - NBUF), NCHW):
        outs[t].wait()


def kernel(logits, save_id, penalty_value, penality_range):
    del penality_range
    tgt = save_id[:, HIST - PRANGE:]
    idx_pad = jnp.concatenate([tgt, tgt[:, : NIDX - PRANGE]], axis=1)
    idx_pad = idx_pad.astype(jnp.int32).reshape(B * NIDX)
    pen16 = jnp.broadcast_to(penalty_value.astype(jnp.float32), (L,))

    run = functools.partial(
        pl.kernel,
        out_type=jax.ShapeDtypeStruct((B, V), jnp.float32),
        mesh=plsc.VectorSubcoreMesh(
            core_axis_name="c", subcore_axis_name="s",
            num_cores=NC, num_subcores=NS,
        ),
        scratch_types=(
            [pltpu.VMEM((RG, WC), jnp.float32)] * NBUF
            + [pltpu.VMEM((RG, TAILC), jnp.float32),
               pltpu.VMEM((RG * NIDX,), jnp.int32),
               pltpu.VMEM((L,), jnp.float32)]
            + [pltpu.SemaphoreType.DMA] * (2 * NBUF)
        ),
        compiler_params=pltpu.CompilerParams(
            needs_layout_passes=False, use_tc_tiling_on_sc=True),
    )(_body)
    return run(logits, idx_pad, pen16)


# speedup vs baseline: 3.1892x; 1.6336x over previous
import functools

import jax
import jax.numpy as jnp
from jax import lax
from jax.experimental import pallas as pl
from jax.experimental.pallas import tpu as pltpu
from jax.experimental.pallas import tpu_sc as plsc

B = 128
V = 100000
N = B * V
HIST = 200
PRANGE = 100
L = 16
NIDX = 112
NKEY = B * NIDX
NC, NS = 2, 16
NW = NC * NS
CHB = 14
CH = 1 << CHB
NFULL = N // CH
TAILW = N - NFULL * CH
NT = 24
NBUF = 3
CAP = NKEY + L


def _filter_chunk(buf, kept, cidx, cvals, pen, c, kept_n, goff):
    lanes = lax.iota(jnp.int32, L)
    nv = (kept_n + L - 1) >> 4

    def body(i, off):
        kv = kept[pl.ds(i * L, L)]
        m = ((i * L + lanes) < kept_n) & ((kv >> CHB) == c)

        def hit(off):
            local = jnp.where(m, kv & (CH - 1), 0)
            g = plsc.load_gather(buf, [local], mask=m)
            plsc.store_compressed(cidx.at[pl.ds(off, L)], kv, mask=m)
            plsc.store_compressed(cvals.at[pl.ds(off, L)], g * pen, mask=m)
            return off + jnp.max(plsc.all_reduce_population_count(m))

        return lax.cond(jnp.any(m), hit, lambda o: o, off)

    return lax.fori_loop(0, nv, body, goff)


def _body(x_hbm, keys_hbm, pen_hbm, o_hbm, *refs):
    bufs = refs[:NBUF]
    keysv, kept, cidx, cvals, penv, padidx, padval = refs[NBUF:NBUF + 7]
    insems = refs[NBUF + 7: 2 * NBUF + 7]
    outsems = refs[2 * NBUF + 7: 3 * NBUF + 7]
    scatsem = refs[3 * NBUF + 7]

    w = lax.axis_index("s") * NC + lax.axis_index("c")
    lanes = lax.iota(jnp.int32, L)

    def chunk_off(t):
        return (w + 32 * t) * CH

    ins = [None] * NT
    for j in range(min(NBUF, NT)):
        ins[j] = pltpu.async_copy(x_hbm.at[pl.ds(chunk_off(j), CH)],
                                  bufs[j], insems[j])
    pltpu.sync_copy(pen_hbm, penv)
    pltpu.sync_copy(keys_hbm, keysv)
    pen = penv[...]

    def scan_body(i, off):
        iv = keysv[pl.ds(i * L, L)]
        m = ((iv >> CHB) & (NW - 1)) == w

        def hit(off):
            plsc.store_compressed(kept.at[pl.ds(off, L)], iv, mask=m)
            return off + jnp.max(plsc.all_reduce_population_count(m))

        return lax.cond(jnp.any(m), hit, lambda o: o, off)

    kept_n = lax.fori_loop(0, NKEY // L, scan_body, 0)

    def flush(fr, to):
        def fbody(r, carry):
            pltpu.async_copy(cvals.at[pl.ds(r * L, L)],
                             o_hbm.at[cidx.at[pl.ds(r * L, L)]], scatsem)
            return carry
        lax.fori_loop(fr, to, fbody, 0)
        return to

    outs = [None] * NT
    goff = jnp.int32(0)
    goffs = []
    fvreg = jnp.int32(0)
    for t in range(NT):
        b = t % NBUF
        ins[t].wait()
        goff = _filter_chunk(bufs[b], kept, cidx, cvals, pen,
                             w + 32 * t, kept_n, goff)
        goffs.append(goff)
        outs[t] = pltpu.async_copy(bufs[b], o_hbm.at[pl.ds(chunk_off(t), CH)],
                                   outsems[b])
        if t >= 1:
            outs[t - 1].wait()
            fvreg = flush(fvreg, goffs[t - 1] >> 4)
            nxt = t + 2
            if nxt < NT:
                ins[nxt] = pltpu.async_copy(
                    x_hbm.at[pl.ds(chunk_off(nxt), CH)], bufs[nxt % NBUF],
                    insems[nxt % NBUF])
    outs[NT - 1].wait()

    ec = w + 32 * NT
    eoff = ec * CH

    @pl.when(w < 13)
    def _efull():
        pltpu.sync_copy(x_hbm.at[pl.ds(eoff, CH)], bufs[0])

    @pl.when(w == 13)
    def _etail_in():
        pltpu.sync_copy(x_hbm.at[pl.ds(NFULL * CH, TAILW)],
                        bufs[0].at[pl.ds(0, TAILW)])

    goff = _filter_chunk(bufs[0], kept, cidx, cvals, pen, ec, kept_n, goff)

    @pl.when(w < 13)
    def _efull_out():
        pltpu.sync_copy(bufs[0], o_hbm.at[pl.ds(eoff, CH)])

    @pl.when(w == 13)
    def _etail_out():
        pltpu.sync_copy(bufs[0].at[pl.ds(0, TAILW)],
                        o_hbm.at[pl.ds(NFULL * CH, TAILW)])

    fvreg = flush(fvreg, goff >> 4)
    rem = goff & (L - 1)
    base = goff - rem

    @pl.when(rem > 0)
    def _remainder():
        kv = cidx[pl.ds(base, L)]
        vv = cvals[pl.ds(base, L)]
        zero16 = jnp.zeros((L,), jnp.int32)
        k0 = kv.at[zero16].get(mode="promise_in_bounds")
        v0 = vv.at[zero16].get(mode="promise_in_bounds")
        mfix = lanes < rem
        padidx[...] = jnp.where(mfix, kv, k0)
        padval[...] = jnp.where(mfix, vv, v0)
        pl.delay(100)
        pltpu.async_copy(padval, o_hbm.at[padidx], scatsem)

    nscat = fvreg + jnp.where(rem > 0, 1, 0)

    def dbody(r, carry):
        pltpu.make_async_copy(cvals.at[pl.ds(0, L)],
                              o_hbm.at[cidx.at[pl.ds(0, L)]], scatsem).wait()
        return carry
    lax.fori_loop(0, nscat, dbody, 0)


def kernel(logits, save_id, penalty_value, penality_range):
    del penality_range
    tgt = save_id[:, HIST - PRANGE:].astype(jnp.int32)
    keys = tgt * B + jnp.arange(B, dtype=jnp.int32)[:, None]
    keys = jnp.concatenate([keys, keys[:, : NIDX - PRANGE]], axis=1)
    keys = keys.reshape(NKEY)
    pen16 = jnp.broadcast_to(penalty_value.astype(jnp.float32), (L,))
    x = logits.T.reshape(N)

    run = functools.partial(
        pl.kernel,
        out_type=jax.ShapeDtypeStruct((N,), jnp.float32),
        mesh=plsc.VectorSubcoreMesh(
            core_axis_name="c", subcore_axis_name="s",
            num_cores=NC, num_subcores=NS,
        ),
        scratch_types=(
            [pltpu.VMEM((CH,), jnp.float32)] * NBUF
            + [pltpu.VMEM((NKEY,), jnp.int32),
               pltpu.VMEM((CAP,), jnp.int32),
               pltpu.VMEM((CAP,), jnp.int32),
               pltpu.VMEM((CAP,), jnp.float32),
               pltpu.VMEM((L,), jnp.float32),
               pltpu.VMEM((L,), jnp.int32),
               pltpu.VMEM((L,), jnp.float32)]
            + [pltpu.SemaphoreType.DMA] * (2 * NBUF + 1)
        ),
        compiler_params=pltpu.CompilerParams(needs_layout_passes=False),
    )(_body)
    return run(x, keys, pen16).reshape(V, B).T
